# baseline (speedup 1.0000x reference)
import jax
import jax.numpy as jnp
from jax import lax
from jax.experimental import pallas as pl
from jax.experimental.pallas import tpu as pltpu
from jax.experimental.pallas import tpu_sc as plsc

NC = 2
NS = 16
L = 16

ROWS = 128
COLS = 32768
S = NC * NS
R = ROWS - S
RB = 32
CB = 2048
NB = COLS // CB
GROUP = 16
GELEMS = GROUP * L
HALF = COLS // 2
NGROUPS_H = HALF // GELEMS


def _sc_argmax_body(logits_hbm, idx_hbm, buf_a, buf_b, idx_buf, sem_l):
    wid = lax.axis_index("s") * NC + lax.axis_index("c")
    row = wid
    lane_iota = lax.iota(jnp.int32, L)

    la = pltpu.async_copy(logits_hbm.at[row, pl.ds(0, HALF)], buf_a, sem_l)
    lb = pltpu.async_copy(logits_hbm.at[row, pl.ds(HALF, HALF)], buf_b, sem_l)

    maxv = jnp.full((L,), -jnp.inf, jnp.float32)
    gidxv = jnp.zeros((L,), jnp.int32)
    gvec0 = jnp.zeros((L,), jnp.int32)
    carry = (maxv, gidxv, gvec0)
    for half, buf, load in ((0, buf_a, la), (1, buf_b, lb)):
        load.wait()

        def gbody(g, c, buf=buf):
            maxv, gidxv, gvec = c
            base = g * GELEMS
            gm = buf[pl.ds(pl.multiple_of(base, L), L)]
            for j in range(1, GROUP):
                gm = jnp.maximum(gm, buf[pl.ds(pl.multiple_of(base + j * L, L), L)])
            m = gm > maxv
            maxv = jnp.maximum(maxv, gm)
            gidxv = jnp.where(m, gvec, gidxv)
            return maxv, gidxv, gvec + 1

        carry = lax.fori_loop(0, NGROUPS_H, gbody, carry)
    maxv, gidxv, _ = carry

    gmax = jnp.max(maxv)
    bgroup = jnp.min(jnp.where(maxv == gmax, gidxv, jnp.int32(2 * NGROUPS_H)))

    in_b = bgroup >= NGROUPS_H
    lgroup = jnp.where(in_b, bgroup - NGROUPS_H, bgroup)
    gbase = lgroup * GELEMS
    best_a = jnp.full((L,), COLS, jnp.int32)
    best_b = jnp.full((L,), COLS, jnp.int32)
    for j in range(GROUP):
        off = pl.multiple_of(gbase + j * L, L)
        idx = gbase + j * L + lane_iota
        va = buf_a[pl.ds(off, L)]
        vb = buf_b[pl.ds(off, L)]
        best_a = jnp.minimum(best_a, jnp.where(va == gmax, idx, jnp.int32(COLS)))
        best_b = jnp.minimum(
            best_b, jnp.where(vb == gmax, idx + HALF, jnp.int32(COLS))
        )
    bidx = jnp.where(in_b, jnp.min(best_b), jnp.min(best_a))

    idx_buf[pl.ds(0, L)] = jnp.zeros((L,), jnp.int32) + bidx
    pltpu.sync_copy(idx_buf, idx_hbm.at[row])


def _sc_argmax(logits):
    mesh = plsc.VectorSubcoreMesh(
        core_axis_name="c", subcore_axis_name="s", num_cores=NC, num_subcores=NS
    )
    run = pl.kernel(
        _sc_argmax_body,
        out_type=jax.ShapeDtypeStruct((S, L), jnp.int32),
        mesh=mesh,
        scratch_types=[
            pltpu.VMEM((HALF,), jnp.float32),
            pltpu.VMEM((HALF,), jnp.float32),
            pltpu.VMEM((L,), jnp.int32),
            pltpu.SemaphoreType.DMA,
        ],
        compiler_params=pltpu.CompilerParams(needs_layout_passes=False),
    )
    return run(logits)


def _tc_argmax_body(x_ref, out_ref, rmax, ridx):
    j = pl.program_id(1)

    @pl.when(j == 0)
    def _():
        rmax[...] = jnp.full((RB, 1), -jnp.inf, jnp.float32)
        ridx[...] = jnp.zeros((RB, 1), jnp.int32)

    blk = x_ref[...]
    bmax = jnp.max(blk, axis=1, keepdims=True)
    iota = lax.broadcasted_iota(jnp.int32, (RB, CB), 1)
    bidx = jnp.min(
        jnp.where(blk == bmax, iota, jnp.int32(CB)), axis=1, keepdims=True
    )
    m = bmax > rmax[...]
    ridx[...] = jnp.where(m, bidx + j * CB, ridx[...])
    rmax[...] = jnp.where(m, bmax, rmax[...])

    @pl.when(j == NB - 1)
    def _():
        out_ref[...] = ridx[...]


def _tc_argmax(logits):
    return pl.pallas_call(
        _tc_argmax_body,
        grid=(R // RB, NB),
        in_specs=[pl.BlockSpec((RB, CB), lambda r, j: (r + S // RB, j))],
        out_specs=pl.BlockSpec((RB, 1), lambda r, j: (r, 0)),
        out_shape=jax.ShapeDtypeStruct((R, 1), jnp.int32),
        scratch_shapes=[
            pltpu.VMEM((RB, 1), jnp.float32),
            pltpu.VMEM((RB, 1), jnp.int32),
        ],
    )(logits)


def _tc_onehot_body(idx_ref, o_ref):
    j = pl.program_id(0)
    iota = lax.broadcasted_iota(jnp.int32, (ROWS, CB), 1) + j * CB
    o_ref[...] = (iota == idx_ref[...]).astype(jnp.float32)


def _tc_onehot(idx_full):
    return pl.pallas_call(
        _tc_onehot_body,
        grid=(NB,),
        in_specs=[pl.BlockSpec((ROWS, 1), lambda j: (0, 0))],
        out_specs=pl.BlockSpec((ROWS, CB), lambda j: (0, j)),
        out_shape=jax.ShapeDtypeStruct((ROWS, COLS), jnp.float32),
    )(idx_full)


@jax.jit
def kernel(logits):
    idx_sc = _sc_argmax(logits)
    idx_tc = _tc_argmax(logits)
    idx_full = jnp.concatenate(
        [jnp.full((S, 1), -1, jnp.int32), idx_tc], axis=0
    )
    base = _tc_onehot(idx_full)
    out = base.at[jnp.arange(S), idx_sc[:, 0]].set(
        1.0, indices_are_sorted=True, unique_indices=True
    )
    return out


# speedup vs baseline: 1.5258x; 1.5258x over previous
import jax
import jax.numpy as jnp
from jax import lax
from jax.experimental import pallas as pl
from jax.experimental.pallas import tpu as pltpu
from jax.experimental.pallas import tpu_sc as plsc

NC = 2
NS = 16
L = 16

ROWS = 128
COLS = 32768
S = 8
R = ROWS - S
RG = 8
TCL = 128
NCHUNK = COLS // TCL
GROUP = 16
GELEMS = GROUP * L
QUARTER = COLS // 4
NGROUPS_Q = QUARTER // GELEMS


def _sc_argmax_body(logits_hbm, tc_idx_hbm, idx_hbm, bufs, idx_buf, sem_l):
    wid = lax.axis_index("s") * NC + lax.axis_index("c")

    @pl.when(wid < S)
    def _():
        row = wid
        lane_iota = lax.iota(jnp.int32, L)
        buf_list = [bufs[0], bufs[1], bufs[2], bufs[3]]

        loads = [
            pltpu.async_copy(
                logits_hbm.at[row, pl.ds(q * QUARTER, QUARTER)], buf_list[q], sem_l
            )
            for q in range(4)
        ]

        maxv = jnp.full((L,), -jnp.inf, jnp.float32)
        gidxv = jnp.zeros((L,), jnp.int32)
        carry = (maxv, gidxv, jnp.zeros((L,), jnp.int32))
        for q in range(4):
            loads[q].wait()
            buf = buf_list[q]

            def gbody(g, c, buf=buf):
                maxv, gidxv, gvec = c
                base = g * GELEMS
                ga = buf[pl.ds(pl.multiple_of(base, L), L)]
                gb = buf[pl.ds(pl.multiple_of(base + 8 * L, L), L)]
                for j in range(1, 8):
                    ga = jnp.maximum(
                        ga, buf[pl.ds(pl.multiple_of(base + j * L, L), L)]
                    )
                    gb = jnp.maximum(
                        gb, buf[pl.ds(pl.multiple_of(base + (8 + j) * L, L), L)]
                    )
                gm = jnp.maximum(ga, gb)
                m = gm > maxv
                maxv = jnp.maximum(maxv, gm)
                gidxv = jnp.where(m, gvec, gidxv)
                return maxv, gidxv, gvec + 1

            carry = lax.fori_loop(0, NGROUPS_Q, gbody, carry)
        maxv, gidxv, _ = carry

        gmax = jnp.max(maxv)
        bgroup = jnp.min(jnp.where(maxv == gmax, gidxv, jnp.int32(4 * NGROUPS_Q)))

        q_of = bgroup // NGROUPS_Q
        lgroup = bgroup - q_of * NGROUPS_Q
        gbase = lgroup * GELEMS
        best = jnp.full((L,), COLS, jnp.int32)
        for q in range(4):
            bq = jnp.full((L,), COLS, jnp.int32)
            for j in range(GROUP):
                off = pl.multiple_of(gbase + j * L, L)
                idx = q * QUARTER + gbase + j * L + lane_iota
                v = buf_list[q][pl.ds(off, L)]
                bq = jnp.minimum(bq, jnp.where(v == gmax, idx, jnp.int32(COLS)))
            best = jnp.where(q_of == q, bq, best)
        bidx = jnp.min(best)

        idx_buf[pl.ds(0, L)] = jnp.zeros((L,), jnp.int32) + bidx
        pltpu.sync_copy(idx_buf, idx_hbm.at[row])


def _sc_argmax(logits, idx_tc):
    mesh = plsc.VectorSubcoreMesh(
        core_axis_name="c", subcore_axis_name="s", num_cores=NC, num_subcores=NS
    )
    run = pl.kernel(
        _sc_argmax_body,
        out_type=jax.ShapeDtypeStruct((S, L), jnp.int32),
        mesh=mesh,
        scratch_types=[
            [pltpu.VMEM((QUARTER,), jnp.float32) for _ in range(4)],
            pltpu.VMEM((L,), jnp.int32),
            pltpu.SemaphoreType.DMA,
        ],
        compiler_params=pltpu.CompilerParams(needs_layout_passes=False),
    )
    return run(logits, idx_tc)


def _tc_argmax_body(x_ref, out_ref):
    lane_iota = lax.broadcasted_iota(jnp.int32, (RG, TCL), 1)
    rmax = jnp.full((RG, TCL), -jnp.inf, jnp.float32)
    cidx = jnp.zeros((RG, TCL), jnp.int32)
    for k in range(NCHUNK):
        chunk = x_ref[:, k * TCL:(k + 1) * TCL]
        m = chunk > rmax
        rmax = jnp.maximum(rmax, chunk)
        cidx = jnp.where(m, jnp.int32(k), cidx)
    gmax = jnp.max(rmax, axis=1, keepdims=True)
    cand = jnp.where(rmax == gmax, cidx * TCL + lane_iota, jnp.int32(COLS))
    out_ref[...] = jnp.min(cand, axis=1, keepdims=True)


def _tc_argmax(logits):
    return pl.pallas_call(
        _tc_argmax_body,
        grid=(R // RG,),
        in_specs=[pl.BlockSpec((RG, COLS), lambda r: (r + S // RG, 0))],
        out_specs=pl.BlockSpec((RG, 1), lambda r: (r, 0)),
        out_shape=jax.ShapeDtypeStruct((R, 1), jnp.int32),
    )(logits)


def _tc_onehot_body(idx_ref, o_ref):
    iota = lax.broadcasted_iota(jnp.int32, (RG, COLS), 1)
    o_ref[...] = (iota == idx_ref[...]).astype(jnp.float32)


def _tc_onehot(idx_tc):
    return pl.pallas_call(
        _tc_onehot_body,
        grid=(R // RG,),
        in_specs=[pl.BlockSpec((RG, 1), lambda r: (r, 0))],
        out_specs=pl.BlockSpec((RG, COLS), lambda r: (r + S // RG, 0)),
        out_shape=jax.ShapeDtypeStruct((ROWS, COLS), jnp.float32),
    )(idx_tc)


def _tc_patch_body(idx_ref, base_ref, o_ref):
    iota = lax.broadcasted_iota(jnp.int32, (S, COLS), 1)
    o_ref[...] = (iota == idx_ref[:, 0:1]).astype(jnp.float32)


def _tc_patch(idx_sc, base):
    return pl.pallas_call(
        _tc_patch_body,
        grid=(1,),
        in_specs=[
            pl.BlockSpec((S, L), lambda r: (0, 0)),
            pl.BlockSpec((8, 128), lambda r: (0, 0)),
        ],
        out_specs=pl.BlockSpec((S, COLS), lambda r: (0, 0)),
        out_shape=jax.ShapeDtypeStruct((ROWS, COLS), jnp.float32),
        input_output_aliases={1: 0},
    )(idx_sc, base)


@jax.jit
def kernel(logits):
    idx_tc = _tc_argmax(logits)
    idx_sc = _sc_argmax(logits, idx_tc)
    base = _tc_onehot(idx_tc)
    return _tc_patch(idx_sc, base)


# speedup vs baseline: 1.7467x; 1.1447x over previous
import jax
import jax.numpy as jnp
from jax import lax
from jax.experimental import pallas as pl
from jax.experimental.pallas import tpu as pltpu
from jax.experimental.pallas import tpu_sc as plsc

NC = 2
NS = 16
L = 16

ROWS = 128
COLS = 32768
S = 16
R = ROWS - S
RG = 8
TCL = 128
NCHUNK = COLS // TCL
GROUP = 16
GELEMS = GROUP * L
QUARTER = COLS // 4
NGROUPS_Q = QUARTER // GELEMS


def _sc_argmax_body(logits_hbm, tc_idx_hbm, idx_hbm, bufs, idx_buf, sem_l):
    wid = lax.axis_index("s") * NC + lax.axis_index("c")

    @pl.when(wid < S)
    def _():
        row = wid
        lane_iota = lax.iota(jnp.int32, L)
        buf_list = [bufs[0], bufs[1], bufs[2], bufs[3]]

        loads = [
            pltpu.async_copy(
                logits_hbm.at[row, pl.ds(q * QUARTER, QUARTER)], buf_list[q], sem_l
            )
            for q in range(4)
        ]

        maxv = jnp.full((L,), -jnp.inf, jnp.float32)
        gidxv = jnp.zeros((L,), jnp.int32)
        carry = (maxv, gidxv, jnp.zeros((L,), jnp.int32))
        for q in range(4):
            loads[q].wait()
            buf = buf_list[q]

            def gbody(g, c, buf=buf):
                maxv, gidxv, gvec = c
                base = g * GELEMS
                ga = buf[pl.ds(pl.multiple_of(base, L), L)]
                gb = buf[pl.ds(pl.multiple_of(base + 8 * L, L), L)]
                for j in range(1, 8):
                    ga = jnp.maximum(
                        ga, buf[pl.ds(pl.multiple_of(base + j * L, L), L)]
                    )
                    gb = jnp.maximum(
                        gb, buf[pl.ds(pl.multiple_of(base + (8 + j) * L, L), L)]
                    )
                gm = jnp.maximum(ga, gb)
                m = gm > maxv
                maxv = jnp.maximum(maxv, gm)
                gidxv = jnp.where(m, gvec, gidxv)
                return maxv, gidxv, gvec + 1

            carry = lax.fori_loop(0, NGROUPS_Q, gbody, carry)
        maxv, gidxv, _ = carry

        gmax = jnp.max(maxv)
        bgroup = jnp.min(jnp.where(maxv == gmax, gidxv, jnp.int32(4 * NGROUPS_Q)))

        q_of = bgroup // NGROUPS_Q
        lgroup = bgroup - q_of * NGROUPS_Q
        gbase = lgroup * GELEMS
        best = jnp.full((L,), COLS, jnp.int32)
        for q in range(4):
            bq = jnp.full((L,), COLS, jnp.int32)
            for j in range(GROUP):
                off = pl.multiple_of(gbase + j * L, L)
                idx = q * QUARTER + gbase + j * L + lane_iota
                v = buf_list[q][pl.ds(off, L)]
                bq = jnp.minimum(bq, jnp.where(v == gmax, idx, jnp.int32(COLS)))
            best = jnp.where(q_of == q, bq, best)
        bidx = jnp.min(best)

        idx_buf[pl.ds(0, L)] = jnp.zeros((L,), jnp.int32) + bidx
        pltpu.sync_copy(idx_buf, idx_hbm.at[row])


def _sc_argmax(logits, idx_tc):
    mesh = plsc.VectorSubcoreMesh(
        core_axis_name="c", subcore_axis_name="s", num_cores=NC, num_subcores=NS
    )
    run = pl.kernel(
        _sc_argmax_body,
        out_type=jax.ShapeDtypeStruct((S, L), jnp.int32),
        mesh=mesh,
        scratch_types=[
            [pltpu.VMEM((QUARTER,), jnp.float32) for _ in range(4)],
            pltpu.VMEM((L,), jnp.int32),
            pltpu.SemaphoreType.DMA,
        ],
        compiler_params=pltpu.CompilerParams(needs_layout_passes=False),
    )
    return run(logits, idx_tc)


def _tc_argmax_body(xl_ref, xr_ref, out_ref):
    lane_iota = lax.broadcasted_iota(jnp.int32, (RG, TCL), 1)
    NH = NCHUNK // 2
    rmaxl = jnp.full((RG, TCL), -jnp.inf, jnp.float32)
    cidxl = jnp.zeros((RG, TCL), jnp.int32)
    rmaxr = jnp.full((RG, TCL), -jnp.inf, jnp.float32)
    cidxr = jnp.zeros((RG, TCL), jnp.int32)
    for k in range(NH):
        cl = xl_ref[:, k * TCL:(k + 1) * TCL]
        cr = xr_ref[:, k * TCL:(k + 1) * TCL]
        ml = cl > rmaxl
        mr = cr > rmaxr
        rmaxl = jnp.maximum(rmaxl, cl)
        rmaxr = jnp.maximum(rmaxr, cr)
        cidxl = jnp.where(ml, jnp.int32(k), cidxl)
        cidxr = jnp.where(mr, jnp.int32(k), cidxr)
    take_r = rmaxr > rmaxl
    rmax = jnp.maximum(rmaxl, rmaxr)
    cidx = jnp.where(take_r, cidxr + NH, cidxl)
    gmax = jnp.max(rmax, axis=1, keepdims=True)
    cand = jnp.where(rmax == gmax, cidx * TCL + lane_iota, jnp.int32(COLS))
    out_ref[...] = jnp.min(cand, axis=1, keepdims=True)


def _tc_argmax(logits):
    return pl.pallas_call(
        _tc_argmax_body,
        grid=(R // RG,),
        in_specs=[
            pl.BlockSpec((RG, COLS // 2), lambda r: (r + S // RG, 0)),
            pl.BlockSpec((RG, COLS // 2), lambda r: (r + S // RG, 1)),
        ],
        out_specs=pl.BlockSpec((RG, 1), lambda r: (r, 0)),
        out_shape=jax.ShapeDtypeStruct((R, 1), jnp.int32),
    )(logits, logits)


RGW = 16


def _tc_onehot_body(idx_ref, o_ref):
    r = pl.program_id(0)
    iota = lax.broadcasted_iota(jnp.int32, (RGW, COLS), 1)
    idx = idx_ref[pl.ds(r * RGW, RGW), :]
    o_ref[...] = (iota == idx).astype(jnp.float32)


def _tc_onehot(idx_tc):
    return pl.pallas_call(
        _tc_onehot_body,
        grid=(R // RGW,),
        in_specs=[pl.BlockSpec((R, 1), lambda r: (0, 0))],
        out_specs=pl.BlockSpec((RGW, COLS), lambda r: (r + S // RGW, 0)),
        out_shape=jax.ShapeDtypeStruct((ROWS, COLS), jnp.float32),
    )(idx_tc)


def _tc_patch_body(idx_ref, base_ref, o_ref):
    iota = lax.broadcasted_iota(jnp.int32, (S, COLS), 1)
    o_ref[...] = (iota == idx_ref[:, 0:1]).astype(jnp.float32)


def _tc_patch(idx_sc, base):
    return pl.pallas_call(
        _tc_patch_body,
        grid=(1,),
        in_specs=[
            pl.BlockSpec((S, L), lambda r: (0, 0)),
            pl.BlockSpec((8, 128), lambda r: (0, 0)),
        ],
        out_specs=pl.BlockSpec((S, COLS), lambda r: (0, 0)),
        out_shape=jax.ShapeDtypeStruct((ROWS, COLS), jnp.float32),
        input_output_aliases={1: 0},
    )(idx_sc, base)


@jax.jit
def kernel(logits):
    idx_tc = _tc_argmax(logits)
    idx_sc = _sc_argmax(logits, idx_tc)
    base = _tc_onehot(idx_tc)
    return _tc_patch(idx_sc, base)


# speedup vs baseline: 1.8219x; 1.0431x over previous
import jax
import jax.numpy as jnp
from jax import lax
from jax.experimental import pallas as pl
from jax.experimental.pallas import tpu as pltpu
from jax.experimental.pallas import tpu_sc as plsc

NC = 2
NS = 16
L = 16

ROWS = 128
COLS = 32768
ROWS_PER_W = ROWS // (NC * NS)
ZCHUNK = 8192
NZ = COLS // ZCHUNK
GROUP = 16
GELEMS = GROUP * L
QUARTER = COLS // 4
NGROUPS_Q = QUARTER // GELEMS


def _hardmax_body(logits_hbm, out_hbm, bufs, zero_buf, patch_buf, sem_z, sem_l):
    wid = lax.axis_index("s") * NC + lax.axis_index("c")
    base_row = wid * ROWS_PER_W
    lane_iota = lax.iota(jnp.int32, L)
    zeros16 = jnp.zeros((L,), jnp.float32)

    def zbody(g, _):
        base = g * (GROUP * L)
        for j in range(GROUP):
            zero_buf[pl.ds(pl.multiple_of(base + j * L, L), L)] = zeros16
        return 0

    lax.fori_loop(0, ZCHUNK // (GROUP * L), zbody, 0)

    def issue_row_loads(r):
        row = base_row + r
        return [
            pltpu.async_copy(
                logits_hbm.at[row, pl.ds(q * QUARTER, QUARTER)],
                bufs[(r % 2) * 4 + q],
                sem_l,
            )
            for q in range(4)
        ]

    loads = [issue_row_loads(0)]

    zcopies = []
    for r in range(ROWS_PER_W):
        row = base_row + r
        for k in range(NZ):
            zcopies.append(
                pltpu.async_copy(
                    zero_buf, out_hbm.at[row, pl.ds(k * ZCHUNK, ZCHUNK)], sem_z
                )
            )

    aligned_offsets = []
    for r in range(ROWS_PER_W):
        if r + 1 < ROWS_PER_W:
            loads.append(issue_row_loads(r + 1))
        buf_list = [bufs[(r % 2) * 4 + q] for q in range(4)]

        maxv = jnp.full((L,), -jnp.inf, jnp.float32)
        gidxv = jnp.zeros((L,), jnp.int32)
        carry = (maxv, gidxv, jnp.zeros((L,), jnp.int32))
        for q in range(4):
            loads[r][q].wait()
            buf = buf_list[q]

            def gbody(g, c, buf=buf):
                maxv, gidxv, gvec = c
                base = g * GELEMS
                ga = buf[pl.ds(pl.multiple_of(base, L), L)]
                gb = buf[pl.ds(pl.multiple_of(base + 8 * L, L), L)]
                for j in range(1, 8):
                    ga = jnp.maximum(
                        ga, buf[pl.ds(pl.multiple_of(base + j * L, L), L)]
                    )
                    gb = jnp.maximum(
                        gb, buf[pl.ds(pl.multiple_of(base + (8 + j) * L, L), L)]
                    )
                gm = jnp.maximum(ga, gb)
                m = gm > maxv
                maxv = jnp.maximum(maxv, gm)
                gidxv = jnp.where(m, gvec, gidxv)
                return maxv, gidxv, gvec + 1

            carry = lax.fori_loop(0, NGROUPS_Q, gbody, carry)
        maxv, gidxv, _ = carry

        gmax = jnp.max(maxv)
        bgroup = jnp.min(jnp.where(maxv == gmax, gidxv, jnp.int32(4 * NGROUPS_Q)))

        q_of = bgroup // NGROUPS_Q
        lgroup = bgroup - q_of * NGROUPS_Q
        gbase = lgroup * GELEMS
        best = jnp.full((L,), COLS, jnp.int32)
        for q in range(4):
            bq = jnp.full((L,), COLS, jnp.int32)
            for j in range(GROUP):
                off = pl.multiple_of(gbase + j * L, L)
                idx = q * QUARTER + gbase + j * L + lane_iota
                v = buf_list[q][pl.ds(off, L)]
                bq = jnp.minimum(bq, jnp.where(v == gmax, idx, jnp.int32(COLS)))
            best = jnp.where(q_of == q, bq, best)
        bidx = jnp.min(best)

        lane = lax.rem(bidx, jnp.int32(L))
        aligned_offsets.append(bidx - lane)
        patch_buf[pl.ds(r * L, L)] = jnp.where(
            lane_iota == lane, jnp.float32(1.0), jnp.float32(0.0)
        )

    for c in zcopies:
        c.wait()
    for r in range(ROWS_PER_W):
        row = base_row + r
        off = pl.multiple_of(aligned_offsets[r], L)
        pltpu.sync_copy(patch_buf.at[pl.ds(r * L, L)], out_hbm.at[row, pl.ds(off, L)])


@jax.jit
def kernel(logits):
    mesh = plsc.VectorSubcoreMesh(
        core_axis_name="c", subcore_axis_name="s", num_cores=NC, num_subcores=NS
    )
    run = pl.kernel(
        _hardmax_body,
        out_type=jax.ShapeDtypeStruct((ROWS, COLS), jnp.float32),
        mesh=mesh,
        scratch_types=[
            [pltpu.VMEM((QUARTER,), jnp.float32) for _ in range(8)],
            pltpu.VMEM((ZCHUNK,), jnp.float32),
            pltpu.VMEM((ROWS_PER_W * L,), jnp.float32),
            pltpu.SemaphoreType.DMA,
            pltpu.SemaphoreType.DMA,
        ],
        compiler_params=pltpu.CompilerParams(needs_layout_passes=False),
    )
    return run(logits)
